# baseline (speedup 1.0000x reference)
import functools
import jax
import jax.numpy as jnp
from jax import lax
from jax.experimental import pallas as pl
from jax.experimental.pallas import tpu as pltpu
from jax.experimental.pallas import tpu_sc as plsc

_H = 8


def _stage_a_body(xt_ref, wqt_ref, wkt_ref, bm_ref, am_ref,
                  qn_ref, kn_ref, srow_ref, s0_ref):
    xt = xt_ref[...]
    q = jnp.dot(xt, wqt_ref[...], preferred_element_type=jnp.float32)
    k = jnp.dot(xt, wkt_ref[...], preferred_element_type=jnp.float32)
    bm = bm_ref[...]
    hq = jnp.dot(q * q, bm, preferred_element_type=jnp.float32)
    hk = jnp.dot(k * k, bm, preferred_element_type=jnp.float32)
    qn = q / (jnp.sqrt(hq) + 1e-8)
    kn = k / (jnp.sqrt(hk) + 1e-8)
    qn_ref[...] = qn
    kn_ref[...] = kn
    n = qn.shape[0]
    ksum = jnp.sum(kn, axis=0, keepdims=True)
    kanch = jnp.sum(kn * am_ref[...], axis=0, keepdims=True)
    srow_ref[...] = (jnp.sum(qn * ksum, axis=1, keepdims=True)
                     + jnp.float32(_H * n)) * 0.5
    s0_ref[...] = (jnp.sum(qn * kanch, axis=1, keepdims=True)
                   + jnp.float32(_H)) / jnp.float32(2 * _H)


def _stage_a(xt, wqt, wkt, bm, amask):
    n, c = xt.shape
    return pl.pallas_call(
        _stage_a_body,
        out_shape=[jax.ShapeDtypeStruct((n, c), jnp.float32),
                   jax.ShapeDtypeStruct((n, c), jnp.float32),
                   jax.ShapeDtypeStruct((n, 1), jnp.float32),
                   jax.ShapeDtypeStruct((n, 1), jnp.float32)],
    )(xt, wqt, wkt, bm, amask)


def _stage_c_body(wave_ref, qn_ref, kn_ref, srow_ref, pred_ref):
    wave = wave_ref[...]
    qn = qn_ref[...]
    kn = kn_ref[...]
    srow = srow_ref[...]
    n = qn.shape[0]
    tmax = jnp.max(wave) + 1
    rows = lax.broadcasted_iota(jnp.int32, (n, 1), 0)
    nf = jnp.float32(n)

    def step(t, acc):
        mf = (wave == t).astype(jnp.float32)
        cb = jnp.sum(mf)
        sr = jnp.sum(mf * srow)
        u = jnp.sum(qn * mf, axis=0, keepdims=True)
        v = jnp.sum(kn * mf, axis=0, keepdims=True)
        uv = jnp.sum(u * v)
        p = (sr - (uv + jnp.float32(_H) * cb * cb) * 0.5) \
            / (jnp.float32(_H) * cb * (nf - cb))
        return acc + jnp.where(rows == t, p, jnp.float32(0.0))

    pred_ref[...] = lax.fori_loop(0, tmax, step,
                                  jnp.zeros((n, 1), jnp.float32))


def _stage_c(wave2d, qn, kn, srow):
    n = qn.shape[0]
    return pl.pallas_call(
        _stage_c_body,
        out_shape=jax.ShapeDtypeStruct((n, 1), jnp.float32),
    )(wave2d, qn, kn, srow)


def _sload(ref, i):
    return ref[pl.ds(i, 16)][0]


def _unpack_nbr(nbr_v, ks, m):
    w = plsc.load_gather(nbr_v, [ks >> 1], mask=m)
    return (w >> ((ks & 1) << 4)) & 0xFFFF


def _make_bfs(np_, npp, pw, qcap):
    mesh = plsc.VectorSubcoreMesh(core_axis_name="c", subcore_axis_name="s")

    @functools.partial(
        pl.kernel, mesh=mesh,
        out_type=[jax.ShapeDtypeStruct((npp,), jnp.int32),
                  jax.ShapeDtypeStruct((qcap,), jnp.int32),
                  jax.ShapeDtypeStruct((16,), jnp.int32)],
        scratch_types=[pltpu.VMEM((pw,), jnp.int32),
                       pltpu.VMEM((npp,), jnp.int32),
                       pltpu.VMEM((npp,), jnp.int32),
                       pltpu.VMEM((qcap,), jnp.int32),
                       pltpu.VMEM((16,), jnp.int32)],
        compiler_params=pltpu.CompilerParams(needs_layout_passes=False),
    )
    def bfs(nbr_hbm, ip_hbm, anc_hbm, wave_hbm, q_hbm, st_hbm,
            nbr_v, ip_v, wave_v, q_v, st_v):
        cid = lax.axis_index("c")
        sid = lax.axis_index("s")

        @pl.when(jnp.logical_and(cid == 0, sid == 0))
        def _():
            pltpu.sync_copy(nbr_hbm, nbr_v)
            pltpu.sync_copy(ip_hbm, ip_v)
            pltpu.sync_copy(anc_hbm, st_v)
            lanes = lax.broadcasted_iota(jnp.int32, (16,), 0)
            lane0 = lanes == 0
            anchor = _sload(st_v, jnp.int32(0))

            allm = lanes >= 0

            def initb(cc, carry):
                plsc.store_scatter(wave_v, [cc * 16 + lanes],
                                   jnp.full((16,), -1, jnp.int32), mask=allm)
                return carry
            lax.fori_loop(0, npp // 16, initb, jnp.int32(0))
            plsc.store_scatter(wave_v, [jnp.full((16,), anchor, jnp.int32)],
                               jnp.zeros((16,), jnp.int32), mask=lane0)
            plsc.store_scatter(q_v, [jnp.zeros((16,), jnp.int32)],
                               jnp.full((16,), anchor, jnp.int32), mask=lane0)

            def wcond(st):
                return st[1] > st[0]

            def wbody(st):
                qhead, qtail, t = st
                tn = t + 1

                def nodeb(p, carry):
                    i = _sload(q_v, p)
                    a0 = _sload(ip_v, i)
                    a1 = _sload(ip_v, i + 1)

                    def chb(cc, c2):
                        k = a0 + cc * 16 + lanes
                        m = k < a1
                        j = _unpack_nbr(nbr_v, jnp.where(m, k, 0), m)
                        j = jnp.where(m, j, 0)
                        wv = plsc.load_gather(wave_v, [j], mask=m)
                        newm = jnp.logical_and(m, wv < 0)
                        plsc.store_scatter(wave_v, [j],
                                           jnp.full((16,), tn, jnp.int32),
                                           mask=newm)
                        return c2
                    lax.fori_loop(0, (a1 - a0 + 15) >> 4, chb, jnp.int32(0))
                    return carry
                lax.fori_loop(qhead, qtail, nodeb, jnp.int32(0))

                def compb(cc, qt):
                    nid = cc * 16 + lanes
                    wv = plsc.load_gather(wave_v, [nid], mask=allm)
                    m = wv == tn
                    mi = m.astype(jnp.int32)
                    cs = plsc.cumsum(mi)
                    plsc.store_scatter(q_v, [qt + cs - mi], nid, mask=m)
                    return qt + jnp.max(cs)
                nqt = lax.fori_loop(0, np_ // 16, compb, qtail)
                return (qtail, nqt, tn)

            fst = lax.while_loop(wcond, wbody,
                                 (jnp.int32(0), jnp.int32(1), jnp.int32(0)))
            st_v[...] = jnp.where(lane0, fst[1], jnp.int32(0))
            pltpu.sync_copy(wave_v, wave_hbm)
            pltpu.sync_copy(q_v, q_hbm)
            pltpu.sync_copy(st_v, st_hbm)

    return bfs


def _make_sweep(npp, pw, qcap):
    mesh = plsc.VectorSubcoreMesh(core_axis_name="c", subcore_axis_name="s")

    @functools.partial(
        pl.kernel, mesh=mesh,
        out_type=jax.ShapeDtypeStruct((npp,), jnp.float32),
        scratch_types=[pltpu.VMEM((pw,), jnp.int32),
                       pltpu.VMEM((npp,), jnp.int32),
                       pltpu.VMEM((npp,), jnp.int32),
                       pltpu.VMEM((qcap,), jnp.int32),
                       pltpu.VMEM((16,), jnp.int32),
                       pltpu.VMEM((npp,), jnp.float32),
                       pltpu.VMEM((npp,), jnp.float32)],
        compiler_params=pltpu.CompilerParams(needs_layout_passes=False),
    )
    def sweep(nbr_hbm, ip_hbm, wave_hbm, q_hbm, st_hbm, s0_hbm, pred_hbm,
              out_hbm, nbr_v, ip_v, wave_v, q_v, st_v, sc_v, pred_v):
        cid = lax.axis_index("c")
        sid = lax.axis_index("s")

        @pl.when(jnp.logical_and(cid == 0, sid == 0))
        def _():
            pltpu.sync_copy(nbr_hbm, nbr_v)
            pltpu.sync_copy(ip_hbm, ip_v)
            pltpu.sync_copy(wave_hbm, wave_v)
            pltpu.sync_copy(q_hbm, q_v)
            pltpu.sync_copy(st_hbm, st_v)
            pltpu.sync_copy(s0_hbm, sc_v)
            pltpu.sync_copy(pred_hbm, pred_v)
            lanes = lax.broadcasted_iota(jnp.int32, (16,), 0)
            lane0 = lanes == 0
            qlen = _sload(st_v, jnp.int32(0))

            def nodeb(p, carry):
                i = _sload(q_v, p)
                t = _sload(wave_v, i)
                a0 = _sload(ip_v, i)
                a1 = _sload(ip_v, i + 1)

                def chb(cc, acc):
                    k = a0 + cc * 16 + lanes
                    m = k < a1
                    j = _unpack_nbr(nbr_v, jnp.where(m, k, 0), m)
                    j = jnp.where(m, j, 0)
                    wv = plsc.load_gather(wave_v, [j], mask=m)
                    valid = m & (wv >= 0) & (wv < t)
                    sj = plsc.load_gather(sc_v, [j], mask=m)
                    return jnp.maximum(
                        acc, jnp.where(valid, sj, jnp.float32(-1e30)))
                acc = lax.fori_loop(0, (a1 - a0 + 15) >> 4, chb,
                                    jnp.full((16,), -1e30, jnp.float32))
                nb = jnp.max(acc)
                nb = jnp.where(nb > jnp.float32(-1e29), nb, jnp.float32(1.0))
                s0i = _sload(sc_v, i)
                pt = _sload(pred_v, t)
                ns = jnp.minimum(jnp.maximum(pt, s0i), nb)
                plsc.store_scatter(sc_v, [jnp.full((16,), i, jnp.int32)],
                                   jnp.full((16,), ns, jnp.float32),
                                   mask=lane0)
                return carry
            lax.fori_loop(0, qlen, nodeb, jnp.int32(0))
            pltpu.sync_copy(sc_v, out_hbm)

    return sweep


def kernel(x, edge_index, anchor, Wq, Wk):
    x = jnp.asarray(x, jnp.float32)
    _, c, n = x.shape
    np_ = ((n + 15) // 16) * 16
    npp = np_ + 16
    qcap = np_ + 32
    e2 = 2 * edge_index.shape[1]
    pw = e2 // 2

    ei = jnp.asarray(edge_index, jnp.int32)
    s = jnp.concatenate([ei[0], ei[1]])
    d = jnp.concatenate([ei[1], ei[0]])
    keys = jnp.sort((jnp.where(s == d, n, s) << 16) | d)
    nbr = keys & 0xFFFF
    src = keys >> 16
    indptr = jnp.searchsorted(
        src, jnp.arange(n + 1, dtype=jnp.int32), side="left").astype(jnp.int32)
    indptr = jnp.pad(indptr, (0, npp - (n + 1)), mode="edge")
    packed = nbr[0::2] | (nbr[1::2] << 16)

    anc = jnp.full((16,), jnp.asarray(anchor, jnp.int32))
    amask = (jnp.arange(n) == jnp.asarray(anchor, jnp.int32)) \
        .astype(jnp.float32).reshape(n, 1)
    dh = c // _H
    bm = jnp.repeat(jnp.repeat(jnp.eye(_H, dtype=jnp.float32), dh, 0), dh, 1)

    xt = x[0].T
    qn, kn, srow, s0 = _stage_a(xt, Wq.astype(jnp.float32).T,
                                Wk.astype(jnp.float32).T, bm, amask)

    wave, queue, stats = _make_bfs(np_, npp, pw, qcap)(packed, indptr, anc)

    pred = _stage_c(wave[:n].reshape(n, 1), qn, kn, srow)

    s0p = jnp.pad(s0.reshape(n), (0, npp - n))
    predp = jnp.pad(pred.reshape(n), (0, npp - n))
    scores = _make_sweep(npp, pw, qcap)(
        packed, indptr, wave, queue, stats, s0p, predp)

    return (x, scores[:n].reshape(1, n, 1))


# speedup vs baseline: 1.5769x; 1.5769x over previous
import functools
import jax
import jax.numpy as jnp
from jax import lax
from jax.experimental import pallas as pl
from jax.experimental.pallas import tpu as pltpu
from jax.experimental.pallas import tpu_sc as plsc

_H = 8


def _stage_a_body(xt_ref, wqt_ref, wkt_ref, bm_ref, am_ref,
                  qn_ref, kn_ref, srow_ref, s0_ref):
    xt = xt_ref[...]
    q = jnp.dot(xt, wqt_ref[...], preferred_element_type=jnp.float32)
    k = jnp.dot(xt, wkt_ref[...], preferred_element_type=jnp.float32)
    bm = bm_ref[...]
    hq = jnp.dot(q * q, bm, preferred_element_type=jnp.float32)
    hk = jnp.dot(k * k, bm, preferred_element_type=jnp.float32)
    qn = q / (jnp.sqrt(hq) + 1e-8)
    kn = k / (jnp.sqrt(hk) + 1e-8)
    qn_ref[...] = qn
    kn_ref[...] = kn
    n = qn.shape[0]
    ksum = jnp.sum(kn, axis=0, keepdims=True)
    kanch = jnp.sum(kn * am_ref[...], axis=0, keepdims=True)
    srow_ref[...] = (jnp.sum(qn * ksum, axis=1, keepdims=True)
                     + jnp.float32(_H * n)) * 0.5
    s0_ref[...] = (jnp.sum(qn * kanch, axis=1, keepdims=True)
                   + jnp.float32(_H)) / jnp.float32(2 * _H)


def _stage_a(xt, wqt, wkt, bm, amask):
    n, c = xt.shape
    return pl.pallas_call(
        _stage_a_body,
        out_shape=[jax.ShapeDtypeStruct((n, c), jnp.float32),
                   jax.ShapeDtypeStruct((n, c), jnp.float32),
                   jax.ShapeDtypeStruct((n, 1), jnp.float32),
                   jax.ShapeDtypeStruct((n, 1), jnp.float32)],
    )(xt, wqt, wkt, bm, amask)


def _stage_c_body(wave_ref, qn_ref, kn_ref, srow_ref, pred_ref):
    wave = wave_ref[...]
    qn = qn_ref[...]
    kn = kn_ref[...]
    srow = srow_ref[...]
    n = qn.shape[0]
    tmax = jnp.max(wave) + 1
    rows = lax.broadcasted_iota(jnp.int32, (n, 1), 0)
    nf = jnp.float32(n)

    def step(t, acc):
        mf = (wave == t).astype(jnp.float32)
        cb = jnp.sum(mf)
        sr = jnp.sum(mf * srow)
        u = jnp.sum(qn * mf, axis=0, keepdims=True)
        v = jnp.sum(kn * mf, axis=0, keepdims=True)
        uv = jnp.sum(u * v)
        p = (sr - (uv + jnp.float32(_H) * cb * cb) * 0.5) \
            / (jnp.float32(_H) * cb * (nf - cb))
        return acc + jnp.where(rows == t, p, jnp.float32(0.0))

    pred_ref[...] = lax.fori_loop(0, tmax, step,
                                  jnp.zeros((n, 1), jnp.float32))


def _stage_c(wave2d, qn, kn, srow):
    n = qn.shape[0]
    return pl.pallas_call(
        _stage_c_body,
        out_shape=jax.ShapeDtypeStruct((n, 1), jnp.float32),
    )(wave2d, qn, kn, srow)


def _sload(ref, i):
    return ref[pl.ds(i, 16)][0]


def _unpack_nbr(nbr_v, ks, m):
    w = plsc.load_gather(nbr_v, [ks >> 1], mask=m)
    return (w >> ((ks & 1) << 4)) & 0xFFFF


def _make_bfs(np_, npp, pw, qcap):
    mesh = plsc.VectorSubcoreMesh(core_axis_name="c", subcore_axis_name="s")

    @functools.partial(
        pl.kernel, mesh=mesh,
        out_type=[jax.ShapeDtypeStruct((npp,), jnp.int32),
                  jax.ShapeDtypeStruct((qcap,), jnp.int32),
                  jax.ShapeDtypeStruct((16,), jnp.int32)],
        scratch_types=[pltpu.VMEM((pw,), jnp.int32),
                       pltpu.VMEM((npp,), jnp.int32),
                       pltpu.VMEM((npp,), jnp.int32),
                       pltpu.VMEM((qcap,), jnp.int32),
                       pltpu.VMEM((16,), jnp.int32)],
        compiler_params=pltpu.CompilerParams(needs_layout_passes=False),
    )
    def bfs(nbr_hbm, ip_hbm, anc_hbm, wave_hbm, q_hbm, st_hbm,
            nbr_v, ip_v, wave_v, q_v, st_v):
        cid = lax.axis_index("c")
        sid = lax.axis_index("s")

        @pl.when(jnp.logical_and(cid == 0, sid == 0))
        def _():
            pltpu.sync_copy(nbr_hbm, nbr_v)
            pltpu.sync_copy(ip_hbm, ip_v)
            pltpu.sync_copy(anc_hbm, st_v)
            lanes = lax.broadcasted_iota(jnp.int32, (16,), 0)
            lane0 = lanes == 0
            anchor = _sload(st_v, jnp.int32(0))

            allm = lanes >= 0

            def initb(cc, carry):
                plsc.store_scatter(wave_v, [cc * 16 + lanes],
                                   jnp.full((16,), -1, jnp.int32), mask=allm)
                return carry
            lax.fori_loop(0, npp // 16, initb, jnp.int32(0))
            plsc.store_scatter(wave_v, [jnp.full((16,), anchor, jnp.int32)],
                               jnp.zeros((16,), jnp.int32), mask=lane0)
            plsc.store_scatter(q_v, [jnp.zeros((16,), jnp.int32)],
                               jnp.full((16,), anchor, jnp.int32), mask=lane0)

            def wcond(st):
                return st[1] > st[0]

            def wbody(st):
                qhead, qtail, t = st
                tn = t + 1

                def nodeb(p, carry):
                    i = _sload(q_v, p)
                    a0 = _sload(ip_v, i)
                    a1 = _sload(ip_v, i + 1)

                    def chb(cc, c2):
                        k = a0 + cc * 16 + lanes
                        m = k < a1
                        j = _unpack_nbr(nbr_v, jnp.where(m, k, 0), m)
                        j = jnp.where(m, j, 0)
                        wv = plsc.load_gather(wave_v, [j], mask=m)
                        newm = jnp.logical_and(m, wv < 0)
                        plsc.store_scatter(wave_v, [j],
                                           jnp.full((16,), tn, jnp.int32),
                                           mask=newm)
                        return c2
                    lax.fori_loop(0, (a1 - a0 + 15) >> 4, chb, jnp.int32(0))
                    return carry
                lax.fori_loop(qhead, qtail, nodeb, jnp.int32(0))

                def compb(cc, qt):
                    nid = cc * 16 + lanes
                    wv = plsc.load_gather(wave_v, [nid], mask=allm)
                    m = wv == tn
                    mi = m.astype(jnp.int32)
                    cs = plsc.cumsum(mi)
                    plsc.store_scatter(q_v, [qt + cs - mi], nid, mask=m)
                    return qt + jnp.max(cs)
                nqt = lax.fori_loop(0, np_ // 16, compb, qtail)
                return (qtail, nqt, tn)

            fst = lax.while_loop(wcond, wbody,
                                 (jnp.int32(0), jnp.int32(1), jnp.int32(0)))
            st_v[...] = jnp.where(lane0, fst[1], jnp.int32(0))
            pltpu.sync_copy(wave_v, wave_hbm)
            pltpu.sync_copy(q_v, q_hbm)
            pltpu.sync_copy(st_v, st_hbm)

    return bfs


def _make_sweep(npp, pw, qcap):
    mesh = plsc.VectorSubcoreMesh(core_axis_name="c", subcore_axis_name="s")

    @functools.partial(
        pl.kernel, mesh=mesh,
        out_type=jax.ShapeDtypeStruct((npp,), jnp.float32),
        scratch_types=[pltpu.VMEM((pw,), jnp.int32),
                       pltpu.VMEM((npp,), jnp.int32),
                       pltpu.VMEM((npp,), jnp.int32),
                       pltpu.VMEM((qcap,), jnp.int32),
                       pltpu.VMEM((16,), jnp.int32),
                       pltpu.VMEM((npp,), jnp.float32),
                       pltpu.VMEM((npp,), jnp.float32)],
        compiler_params=pltpu.CompilerParams(needs_layout_passes=False),
    )
    def sweep(nbr_hbm, ip_hbm, wave_hbm, q_hbm, st_hbm, s0_hbm, pred_hbm,
              out_hbm, nbr_v, ip_v, wave_v, q_v, st_v, sc_v, pred_v):
        cid = lax.axis_index("c")
        sid = lax.axis_index("s")

        @pl.when(jnp.logical_and(cid == 0, sid == 0))
        def _():
            pltpu.sync_copy(nbr_hbm, nbr_v)
            pltpu.sync_copy(ip_hbm, ip_v)
            pltpu.sync_copy(wave_hbm, wave_v)
            pltpu.sync_copy(q_hbm, q_v)
            pltpu.sync_copy(st_hbm, st_v)
            pltpu.sync_copy(s0_hbm, sc_v)
            pltpu.sync_copy(pred_hbm, pred_v)
            lanes = lax.broadcasted_iota(jnp.int32, (16,), 0)
            lane0 = lanes == 0
            qlen = _sload(st_v, jnp.int32(0))

            def nodeb(p, carry):
                i = _sload(q_v, p)
                t = _sload(wave_v, i)
                a0 = _sload(ip_v, i)
                a1 = _sload(ip_v, i + 1)

                def chb(cc, acc):
                    k = a0 + cc * 16 + lanes
                    m = k < a1
                    j = _unpack_nbr(nbr_v, jnp.where(m, k, 0), m)
                    j = jnp.where(m, j, 0)
                    wv = plsc.load_gather(wave_v, [j], mask=m)
                    valid = m & (wv >= 0) & (wv < t)
                    sj = plsc.load_gather(sc_v, [j], mask=m)
                    return jnp.maximum(
                        acc, jnp.where(valid, sj, jnp.float32(-1e30)))
                acc = lax.fori_loop(0, (a1 - a0 + 15) >> 4, chb,
                                    jnp.full((16,), -1e30, jnp.float32))
                nb = jnp.max(acc)
                nb = jnp.where(nb > jnp.float32(-1e29), nb, jnp.float32(1.0))
                s0i = _sload(sc_v, i)
                pt = _sload(pred_v, t)
                ns = jnp.minimum(jnp.maximum(pt, s0i), nb)
                plsc.store_scatter(sc_v, [jnp.full((16,), i, jnp.int32)],
                                   jnp.full((16,), ns, jnp.float32),
                                   mask=lane0)
                return carry
            lax.fori_loop(0, qlen, nodeb, jnp.int32(0))
            pltpu.sync_copy(sc_v, out_hbm)

    return sweep


def kernel(x, edge_index, anchor, Wq, Wk):
    x = jnp.asarray(x, jnp.float32)
    _, c, n = x.shape
    np_ = ((n + 15) // 16) * 16
    npp = np_ + 16
    qcap = np_ + 32
    e2 = 2 * edge_index.shape[1]
    pw = e2 // 2

    ei = jnp.asarray(edge_index, jnp.int32)
    s = jnp.concatenate([ei[0], ei[1]])
    d = jnp.concatenate([ei[1], ei[0]])
    sk = jnp.where(s == d, n, s)
    keys = jnp.sort((sk << 16) | d)
    nbr = keys & 0xFFFF
    deg = jnp.zeros((n + 1,), jnp.int32).at[sk].add(1)
    indptr = jnp.concatenate(
        [jnp.zeros((1,), jnp.int32), jnp.cumsum(deg[:n], dtype=jnp.int32)])
    indptr = jnp.pad(indptr, (0, npp - (n + 1)), mode="edge")
    packed = nbr[0::2] | (nbr[1::2] << 16)

    anc = jnp.full((16,), jnp.asarray(anchor, jnp.int32))
    amask = (jnp.arange(n) == jnp.asarray(anchor, jnp.int32)) \
        .astype(jnp.float32).reshape(n, 1)
    dh = c // _H
    bm = jnp.repeat(jnp.repeat(jnp.eye(_H, dtype=jnp.float32), dh, 0), dh, 1)

    xt = x[0].T
    qn, kn, srow, s0 = _stage_a(xt, Wq.astype(jnp.float32).T,
                                Wk.astype(jnp.float32).T, bm, amask)

    wave, queue, stats = _make_bfs(np_, npp, pw, qcap)(packed, indptr, anc)

    pred = _stage_c(wave[:n].reshape(n, 1), qn, kn, srow)

    s0p = jnp.pad(s0.reshape(n), (0, npp - n))
    predp = jnp.pad(pred.reshape(n), (0, npp - n))
    scores = _make_sweep(npp, pw, qcap)(
        packed, indptr, wave, queue, stats, s0p, predp)

    return (x, scores[:n].reshape(1, n, 1))


# speedup vs baseline: 2.3287x; 1.4767x over previous
import functools
import jax
import jax.numpy as jnp
from jax import lax
from jax.experimental import pallas as pl
from jax.experimental.pallas import tpu as pltpu
from jax.experimental.pallas import tpu_sc as plsc

_H = 8


def _stage_a_body(xt_ref, wqt_ref, wkt_ref, bm_ref, am_ref,
                  qn_ref, kn_ref, srow_ref, s0_ref):
    xt = xt_ref[...]
    q = jnp.dot(xt, wqt_ref[...], preferred_element_type=jnp.float32)
    k = jnp.dot(xt, wkt_ref[...], preferred_element_type=jnp.float32)
    bm = bm_ref[...]
    hq = jnp.dot(q * q, bm, preferred_element_type=jnp.float32)
    hk = jnp.dot(k * k, bm, preferred_element_type=jnp.float32)
    qn = q / (jnp.sqrt(hq) + 1e-8)
    kn = k / (jnp.sqrt(hk) + 1e-8)
    qn_ref[...] = qn
    kn_ref[...] = kn
    n = qn.shape[0]
    ksum = jnp.sum(kn, axis=0, keepdims=True)
    kanch = jnp.sum(kn * am_ref[...], axis=0, keepdims=True)
    srow_ref[...] = (jnp.sum(qn * ksum, axis=1, keepdims=True)
                     + jnp.float32(_H * n)) * 0.5
    s0_ref[...] = (jnp.sum(qn * kanch, axis=1, keepdims=True)
                   + jnp.float32(_H)) / jnp.float32(2 * _H)


def _stage_a(xt, wqt, wkt, bm, amask):
    n, c = xt.shape
    return pl.pallas_call(
        _stage_a_body,
        out_shape=[jax.ShapeDtypeStruct((n, c), jnp.float32),
                   jax.ShapeDtypeStruct((n, c), jnp.float32),
                   jax.ShapeDtypeStruct((n, 1), jnp.float32),
                   jax.ShapeDtypeStruct((n, 1), jnp.float32)],
    )(xt, wqt, wkt, bm, amask)


def _stage_c_body(wave_ref, qn_ref, kn_ref, srow_ref, pred_ref):
    wave = wave_ref[...]
    qn = qn_ref[...]
    kn = kn_ref[...]
    srow = srow_ref[...]
    n = qn.shape[0]
    tmax = jnp.max(wave) + 1
    rows = lax.broadcasted_iota(jnp.int32, (n, 1), 0)
    nf = jnp.float32(n)

    def step(t, acc):
        mf = (wave == t).astype(jnp.float32)
        cb = jnp.sum(mf)
        sr = jnp.sum(mf * srow)
        u = jnp.sum(qn * mf, axis=0, keepdims=True)
        v = jnp.sum(kn * mf, axis=0, keepdims=True)
        uv = jnp.sum(u * v)
        p = (sr - (uv + jnp.float32(_H) * cb * cb) * 0.5) \
            / (jnp.float32(_H) * cb * (nf - cb))
        return acc + jnp.where(rows == t, p, jnp.float32(0.0))

    pred_ref[...] = lax.fori_loop(0, tmax, step,
                                  jnp.zeros((n, 1), jnp.float32))


def _stage_c(wave2d, qn, kn, srow):
    n = qn.shape[0]
    return pl.pallas_call(
        _stage_c_body,
        out_shape=jax.ShapeDtypeStruct((n, 1), jnp.float32),
    )(wave2d, qn, kn, srow)


def _sload(ref, i):
    return ref[pl.ds(i, 16)][0]


def _unpack_nbr(nbr_v, ks, m):
    w = plsc.load_gather(nbr_v, [ks >> 1], mask=m)
    return (w >> ((ks & 1) << 4)) & 0xFFFF


def _make_bfs(np_, npp, pw, qcap):
    mesh = plsc.VectorSubcoreMesh(core_axis_name="c", subcore_axis_name="s")

    @functools.partial(
        pl.kernel, mesh=mesh,
        out_type=[jax.ShapeDtypeStruct((npp,), jnp.int32),
                  jax.ShapeDtypeStruct((qcap,), jnp.int32),
                  jax.ShapeDtypeStruct((16,), jnp.int32),
                  jax.ShapeDtypeStruct((qcap,), jnp.int32)],
        scratch_types=[pltpu.VMEM((pw,), jnp.int32),
                       pltpu.VMEM((npp,), jnp.int32),
                       pltpu.VMEM((npp,), jnp.int32),
                       pltpu.VMEM((qcap,), jnp.int32),
                       pltpu.VMEM((16,), jnp.int32),
                       pltpu.VMEM((qcap,), jnp.int32)],
        compiler_params=pltpu.CompilerParams(needs_layout_passes=False),
    )
    def bfs(nbr_hbm, ip_hbm, anc_hbm, wave_hbm, q_hbm, st_hbm, woff_hbm,
            nbr_v, ip_v, wave_v, q_v, st_v, woff_v):
        cid = lax.axis_index("c")
        sid = lax.axis_index("s")

        @pl.when(jnp.logical_and(cid == 0, sid == 0))
        def _():
            pltpu.sync_copy(nbr_hbm, nbr_v)
            pltpu.sync_copy(ip_hbm, ip_v)
            pltpu.sync_copy(anc_hbm, st_v)
            lanes = lax.broadcasted_iota(jnp.int32, (16,), 0)
            lane0 = lanes == 0
            anchor = _sload(st_v, jnp.int32(0))

            allm = lanes >= 0

            def initb(cc, carry):
                plsc.store_scatter(wave_v, [cc * 16 + lanes],
                                   jnp.full((16,), -1, jnp.int32), mask=allm)
                return carry
            lax.fori_loop(0, npp // 16, initb, jnp.int32(0))
            plsc.store_scatter(wave_v, [jnp.full((16,), anchor, jnp.int32)],
                               jnp.zeros((16,), jnp.int32), mask=lane0)
            plsc.store_scatter(q_v, [jnp.zeros((16,), jnp.int32)],
                               jnp.full((16,), anchor, jnp.int32), mask=lane0)

            def wcond(st):
                return st[1] > st[0]

            def wbody(st):
                qhead, qtail, t = st
                tn = t + 1
                plsc.store_scatter(woff_v, [jnp.full((16,), t, jnp.int32)],
                                   jnp.full((16,), qhead, jnp.int32),
                                   mask=lane0)

                def nodeb(p, carry):
                    i = _sload(q_v, p)
                    a0 = _sload(ip_v, i)
                    a1 = _sload(ip_v, i + 1)

                    def chb(cc, c2):
                        k = a0 + cc * 16 + lanes
                        m = k < a1
                        j = _unpack_nbr(nbr_v, jnp.where(m, k, 0), m)
                        j = jnp.where(m, j, 0)
                        wv = plsc.load_gather(wave_v, [j], mask=m)
                        newm = jnp.logical_and(m, wv < 0)
                        plsc.store_scatter(wave_v, [j],
                                           jnp.full((16,), tn, jnp.int32),
                                           mask=newm)
                        return c2
                    lax.fori_loop(0, (a1 - a0 + 15) >> 4, chb, jnp.int32(0))
                    return carry
                lax.fori_loop(qhead, qtail, nodeb, jnp.int32(0))

                def compb(cc, qt):
                    nid = cc * 16 + lanes
                    wv = plsc.load_gather(wave_v, [nid], mask=allm)
                    m = wv == tn
                    mi = m.astype(jnp.int32)
                    cs = plsc.cumsum(mi)
                    plsc.store_scatter(q_v, [qt + cs - mi], nid, mask=m)
                    return qt + jnp.max(cs)
                nqt = lax.fori_loop(0, np_ // 16, compb, qtail)
                return (qtail, nqt, tn)

            fst = lax.while_loop(wcond, wbody,
                                 (jnp.int32(0), jnp.int32(1), jnp.int32(0)))
            plsc.store_scatter(woff_v, [jnp.full((16,), fst[2], jnp.int32)],
                               jnp.full((16,), fst[1], jnp.int32), mask=lane0)
            plsc.store_scatter(woff_v,
                               [jnp.full((16,), qcap - 16, jnp.int32)],
                               jnp.full((16,), fst[2], jnp.int32), mask=lane0)
            st_v[...] = jnp.where(lane0, fst[1], jnp.int32(0))
            pltpu.sync_copy(wave_v, wave_hbm)
            pltpu.sync_copy(q_v, q_hbm)
            pltpu.sync_copy(st_v, st_hbm)
            pltpu.sync_copy(woff_v, woff_hbm)

    return bfs


_ROW = 512


def _make_sweep(npp, pw, qcap):
    mesh = plsc.VectorSubcoreMesh(core_axis_name="c", subcore_axis_name="s")

    @functools.partial(
        pl.kernel, mesh=mesh,
        out_type=jax.ShapeDtypeStruct((npp,), jnp.float32),
        scratch_types=[pltpu.VMEM((pw,), jnp.int32),
                       pltpu.VMEM((npp,), jnp.int32),
                       pltpu.VMEM((npp,), jnp.int32),
                       pltpu.VMEM((qcap,), jnp.int32),
                       pltpu.VMEM((qcap,), jnp.int32),
                       pltpu.VMEM((npp,), jnp.float32),
                       pltpu.VMEM((npp,), jnp.float32),
                       pltpu.VMEM((_ROW,), jnp.float32),
                       pltpu.VMEM((16, _ROW), jnp.float32),
                       pltpu.VMEM_SHARED((16, _ROW), jnp.float32)],
        compiler_params=pltpu.CompilerParams(needs_layout_passes=False),
    )
    def sweep(nbr_hbm, ip_hbm, wave_hbm, q_hbm, woff_hbm, s0_hbm, pred_hbm,
              out_hbm, nbr_v, ip_v, wave_v, q_v, woff_v, sc_v, pred_v,
              snew_v, rbuf_v, shared):
        cid = lax.axis_index("c")
        sid = lax.axis_index("s")

        @pl.when(cid == 0)
        def _():
            pltpu.sync_copy(nbr_hbm, nbr_v)
            pltpu.sync_copy(ip_hbm, ip_v)
            pltpu.sync_copy(wave_hbm, wave_v)
            pltpu.sync_copy(q_hbm, q_v)
            pltpu.sync_copy(woff_hbm, woff_v)
            pltpu.sync_copy(s0_hbm, sc_v)
            pltpu.sync_copy(pred_hbm, pred_v)
            lanes = lax.broadcasted_iota(jnp.int32, (16,), 0)
            lane0 = lanes == 0
            nwaves = _sload(woff_v, jnp.int32(qcap - 16))

            def wavef(w, carry):
                qs = _sload(woff_v, w)
                qe = _sload(woff_v, w + 1)
                size = qe - qs
                chunk = (size + 15) >> 4
                base = qs + sid * chunk
                myn = jnp.clip(qe - base, 0, chunk)
                pt = _sload(pred_v, w)

                def nodeb(mm, c):
                    i = _sload(q_v, base + mm)
                    t = _sload(wave_v, i)
                    a0 = _sload(ip_v, i)
                    a1 = _sload(ip_v, i + 1)

                    def chb(cc, acc):
                        k = a0 + cc * 16 + lanes
                        m = k < a1
                        j = _unpack_nbr(nbr_v, jnp.where(m, k, 0), m)
                        j = jnp.where(m, j, 0)
                        wv = plsc.load_gather(wave_v, [j], mask=m)
                        valid = m & (wv >= 0) & (wv < t)
                        sj = plsc.load_gather(sc_v, [j], mask=m)
                        return jnp.maximum(
                            acc, jnp.where(valid, sj, jnp.float32(-1e30)))
                    acc = lax.fori_loop(0, (a1 - a0 + 15) >> 4, chb,
                                        jnp.full((16,), -1e30, jnp.float32))
                    nb = jnp.max(acc)
                    nb = jnp.where(nb > jnp.float32(-1e29), nb,
                                   jnp.float32(1.0))
                    s0i = _sload(sc_v, i)
                    ns = jnp.minimum(jnp.maximum(pt, s0i), nb)
                    plsc.store_scatter(snew_v,
                                       [jnp.full((16,), mm, jnp.int32)],
                                       jnp.full((16,), ns, jnp.float32),
                                       mask=lane0)
                    return c
                lax.fori_loop(0, myn, nodeb, jnp.int32(0))
                pltpu.sync_copy(snew_v, shared.at[sid])
                plsc.subcore_barrier()
                pltpu.sync_copy(shared, rbuf_v)
                plsc.subcore_barrier()
                for kk in range(16):
                    bk = qs + kk * chunk
                    ck = jnp.clip(qe - bk, 0, chunk)

                    def apb(b, c, kk=kk, bk=bk, ck=ck):
                        off = b * 16
                        ids = q_v[pl.ds(bk + off, 16)]
                        vals = rbuf_v[kk, pl.ds(off, 16)]
                        m = (off + lanes) < ck
                        plsc.store_scatter(sc_v, [jnp.where(m, ids, 0)],
                                           vals, mask=m)
                        return c
                    lax.fori_loop(0, (ck + 15) >> 4, apb, jnp.int32(0))
                return carry
            lax.fori_loop(0, nwaves, wavef, jnp.int32(0))

            @pl.when(sid == 0)
            def _():
                pltpu.sync_copy(sc_v, out_hbm)

    return sweep


def kernel(x, edge_index, anchor, Wq, Wk):
    x = jnp.asarray(x, jnp.float32)
    _, c, n = x.shape
    np_ = ((n + 15) // 16) * 16
    npp = np_ + 16
    qcap = np_ + 32
    e2 = 2 * edge_index.shape[1]
    pw = e2 // 2

    ei = jnp.asarray(edge_index, jnp.int32)
    s = jnp.concatenate([ei[0], ei[1]])
    d = jnp.concatenate([ei[1], ei[0]])
    sk = jnp.where(s == d, n, s)
    keys = jnp.sort((sk << 16) | d)
    nbr = keys & 0xFFFF
    deg = jnp.zeros((n + 1,), jnp.int32).at[sk].add(1)
    indptr = jnp.concatenate(
        [jnp.zeros((1,), jnp.int32), jnp.cumsum(deg[:n], dtype=jnp.int32)])
    indptr = jnp.pad(indptr, (0, npp - (n + 1)), mode="edge")
    packed = nbr[0::2] | (nbr[1::2] << 16)

    anc = jnp.full((16,), jnp.asarray(anchor, jnp.int32))
    amask = (jnp.arange(n) == jnp.asarray(anchor, jnp.int32)) \
        .astype(jnp.float32).reshape(n, 1)
    dh = c // _H
    bm = jnp.repeat(jnp.repeat(jnp.eye(_H, dtype=jnp.float32), dh, 0), dh, 1)

    xt = x[0].T
    qn, kn, srow, s0 = _stage_a(xt, Wq.astype(jnp.float32).T,
                                Wk.astype(jnp.float32).T, bm, amask)

    wave, queue, stats, woff = _make_bfs(np_, npp, pw, qcap)(
        packed, indptr, anc)

    pred = _stage_c(wave[:n].reshape(n, 1), qn, kn, srow)

    s0p = jnp.pad(s0.reshape(n), (0, npp - n))
    predp = jnp.pad(pred.reshape(n), (0, npp - n))
    scores = _make_sweep(npp, pw, qcap)(
        packed, indptr, wave, queue, woff, s0p, predp)

    return (x, scores[:n].reshape(1, n, 1))
